# baseline (speedup 1.0000x reference)
import jax
import jax.numpy as jnp
from jax.experimental import pallas as pl
from jax.experimental.pallas import tpu as pltpu

_N = 10000
_D = 128
_CH = 200
_NBUF = 2
_SPLITS = (0, 96, 200)
_NCHUNK = _N // _CH


def _start_chunk(adj_hbm, buf_ref, sem, chunk, slot):
    for p in range(len(_SPLITS) - 1):
        lo, hi = _SPLITS[p], _SPLITS[p + 1]
        pltpu.make_async_copy(
            adj_hbm.at[pl.ds(chunk * _CH + lo, hi - lo), :],
            buf_ref.at[slot, pl.ds(lo, hi - lo), :],
            sem.at[slot, p],
        ).start()


def _wait_chunk(adj_hbm, buf_ref, sem, chunk, slot):
    for p in range(len(_SPLITS) - 1):
        lo, hi = _SPLITS[p], _SPLITS[p + 1]
        pltpu.make_async_copy(
            adj_hbm.at[pl.ds(chunk * _CH + lo, hi - lo), :],
            buf_ref.at[slot, pl.ds(lo, hi - lo), :],
            sem.at[slot, p],
        ).wait()


def _body(modal_ref, adj_hbm, feature_ref, w_ref, b_ref, out_ref,
          buf_ref, support_ref, sem):
    i = pl.program_id(0)
    slot = jax.lax.rem(i, _NBUF)

    @pl.when(i == 0)
    def _prologue():
        for s in range(_NBUF):
            _start_chunk(adj_hbm, buf_ref, sem, s, s)
        support_ref[:] = jnp.dot(feature_ref[:], w_ref[:],
                                 preferred_element_type=jnp.float32)

    _wait_chunk(adj_hbm, buf_ref, sem, i, slot)

    acc = jnp.dot(buf_ref[slot], support_ref[:],
                  preferred_element_type=jnp.float32)
    heter = acc + b_ref[:]
    feat_blk = feature_ref[pl.ds(i * _CH, _CH), :]
    out_ref[:] = jnp.where(modal_ref[0] > 1, heter, feat_blk)

    nxt = i + _NBUF

    @pl.when(nxt < _NCHUNK)
    def _refill():
        _start_chunk(adj_hbm, buf_ref, sem, nxt, slot)


def kernel(feature, num_modal, adj_weight, W, b):
    feature = feature.astype(jnp.float32)
    modal = jnp.asarray(num_modal, jnp.int32).reshape(1)
    b2 = b.reshape(1, _D)

    grid_spec = pltpu.PrefetchScalarGridSpec(
        num_scalar_prefetch=1,
        grid=(_NCHUNK,),
        in_specs=[
            pl.BlockSpec(memory_space=pl.ANY),
            pl.BlockSpec((_N, _D), lambda i, modal_ref: (0, 0)),
            pl.BlockSpec((_D, _D), lambda i, modal_ref: (0, 0)),
            pl.BlockSpec((1, _D), lambda i, modal_ref: (0, 0)),
        ],
        out_specs=pl.BlockSpec((_CH, _D), lambda i, modal_ref: (i, 0)),
        scratch_shapes=[
            pltpu.VMEM((_NBUF, _CH, _N), jnp.float32),
            pltpu.VMEM((_N, _D), jnp.float32),
            pltpu.SemaphoreType.DMA((_NBUF, len(_SPLITS) - 1)),
        ],
    )

    out = pl.pallas_call(
        _body,
        grid_spec=grid_spec,
        out_shape=jax.ShapeDtypeStruct((_N, _D), jnp.float32),
        compiler_params=pltpu.CompilerParams(
            dimension_semantics=("arbitrary",),
        ),
    )(modal, adj_weight, feature, W, b2)
    return out


# speedup vs baseline: 1.0031x; 1.0031x over previous
import jax
import jax.numpy as jnp
from jax.experimental import pallas as pl
from jax.experimental.pallas import tpu as pltpu

_N = 10000
_D = 128
_CH = 200
_NBUF = 2
_NCHUNK = _N // _CH


def _copy(adj_hbm, buf_ref, sem, chunk, slot):
    return pltpu.make_async_copy(
        adj_hbm.at[pl.ds(chunk * _CH, _CH), :],
        buf_ref.at[slot],
        sem.at[slot],
    )


def _body(modal_ref, adj_hbm, feature_ref, w_ref, b_ref, out_ref,
          buf_ref, support_ref, sem):
    i = pl.program_id(0)
    slot = jax.lax.rem(i, _NBUF)

    @pl.when(i == 0)
    def _prologue():
        for s in range(_NBUF):
            _copy(adj_hbm, buf_ref, sem, s, s).start()
        support_ref[:] = jnp.dot(feature_ref[:], w_ref[:],
                                 preferred_element_type=jnp.float32)

    _copy(adj_hbm, buf_ref, sem, i, slot).wait()

    acc = jnp.dot(buf_ref[slot], support_ref[:],
                  preferred_element_type=jnp.float32)
    heter = acc + b_ref[:]
    feat_blk = feature_ref[pl.ds(i * _CH, _CH), :]
    out_ref[:] = jnp.where(modal_ref[0] > 1, heter, feat_blk)

    nxt = i + _NBUF

    @pl.when(nxt < _NCHUNK)
    def _refill():
        _copy(adj_hbm, buf_ref, sem, nxt, slot).start()


def kernel(feature, num_modal, adj_weight, W, b):
    feature = feature.astype(jnp.float32)
    modal = jnp.asarray(num_modal, jnp.int32).reshape(1)
    b2 = b.reshape(1, _D)

    grid_spec = pltpu.PrefetchScalarGridSpec(
        num_scalar_prefetch=1,
        grid=(_NCHUNK,),
        in_specs=[
            pl.BlockSpec(memory_space=pl.ANY),
            pl.BlockSpec((_N, _D), lambda i, modal_ref: (0, 0)),
            pl.BlockSpec((_D, _D), lambda i, modal_ref: (0, 0)),
            pl.BlockSpec((1, _D), lambda i, modal_ref: (0, 0)),
        ],
        out_specs=pl.BlockSpec((_CH, _D), lambda i, modal_ref: (i, 0)),
        scratch_shapes=[
            pltpu.VMEM((_NBUF, _CH, _N), jnp.float32),
            pltpu.VMEM((_N, _D), jnp.float32),
            pltpu.SemaphoreType.DMA((_NBUF,)),
        ],
    )

    out = pl.pallas_call(
        _body,
        grid_spec=grid_spec,
        out_shape=jax.ShapeDtypeStruct((_N, _D), jnp.float32),
        compiler_params=pltpu.CompilerParams(
            dimension_semantics=("arbitrary",),
        ),
    )(modal, adj_weight, feature, W, b2)
    return out
